# baseline (speedup 1.0000x reference)
import functools

import jax
import jax.numpy as jnp
from jax import lax
from jax.experimental import pallas as pl
from jax.experimental.pallas import tpu as pltpu
from jax.experimental.pallas import tpu_sc as plsc

HIDDEN = 512
OUTPUTS = 2
NUM_GRAPHS = 64
EPS = 1e-5

NW = 32
C_SC = 160
SC_ROWS = NW * C_SC
S_TC = 10000 - SC_ROWS
BLK = S_TC // 2
CHUNK = 32
NCH = C_SC // CHUNK
LANES = 16
NCOL = HIDDEN // LANES
OPAD = 128


def _tc_sum_kernel(batch_ref, x_ref, acc_ref, cnt_ref):
    i = pl.program_id(0)

    @pl.when(i == 0)
    def _init():
        acc_ref[...] = jnp.zeros_like(acc_ref)
        cnt_ref[...] = jnp.zeros_like(cnt_ref)

    b = batch_ref[0, 0, :]
    gids = lax.broadcasted_iota(jnp.int32, (NUM_GRAPHS, BLK), 0)
    onehot = (b[None, :] == gids).astype(jnp.float32)
    acc_ref[...] += jnp.dot(onehot, x_ref[...],
                            preferred_element_type=jnp.float32)
    cnt_ref[...] = cnt_ref[...] + jnp.sum(onehot, axis=1, keepdims=True)


def _sc_sum_kernel(x_hbm, batch_hbm, acc_out, cnt_out,
                   rows_v, acc_v, cnt_v, idx_v, sem0, sem1):
    c = lax.axis_index("c")
    s = lax.axis_index("s")
    wid = c * 16 + s
    base = S_TC + wid * C_SC

    pltpu.sync_copy(batch_hbm.at[pl.ds(base, C_SC)], idx_v)

    zero = jnp.zeros((LANES,), jnp.float32)

    def zrow(g, carry):
        for k in range(NCOL):
            acc_v[g, pl.ds(k * LANES, LANES)] = zero
        cnt_v[g, :] = zero
        return carry

    lax.fori_loop(0, NUM_GRAPHS, zrow, 0)

    sems = [sem0, sem1]
    copies = [None, None]
    copies[0] = pltpu.make_async_copy(
        x_hbm.at[pl.ds(base, CHUNK)], rows_v.at[0], sems[0])
    copies[0].start()
    ones = jnp.ones((LANES,), jnp.float32)

    for ch in range(NCH):
        buf = ch % 2
        if ch + 1 < NCH:
            nbuf = (ch + 1) % 2
            copies[nbuf] = pltpu.make_async_copy(
                x_hbm.at[pl.ds(base + (ch + 1) * CHUNK, CHUNK)],
                rows_v.at[nbuf], sems[nbuf])
            copies[nbuf].start()
        copies[buf].wait()

        def group_body(g, carry, ch=ch, buf=buf):
            ids = idx_v[pl.ds(ch * CHUNK + g * LANES, LANES)]
            r0 = g * LANES
            for j in range(LANES):
                bj = ids[j]
                for k in range(NCOL):
                    plsc.addupdate(
                        acc_v.at[bj, pl.ds(k * LANES, LANES)],
                        rows_v[buf, r0 + j, pl.ds(k * LANES, LANES)])
                plsc.addupdate(cnt_v.at[bj, :], ones)
            return carry

        lax.fori_loop(0, CHUNK // LANES, group_body, 0)

    pltpu.sync_copy(acc_v, acc_out.at[wid])
    pltpu.sync_copy(cnt_v, cnt_out.at[wid])


def _mlp_kernel(acc_tc_ref, cnt_tc_ref, acc_sc_ref, cnt_sc_ref,
                w1_ref, b1_ref, gamma_ref, beta_ref, w2t_ref, b2_ref, o_ref):
    sums = acc_tc_ref[...] + jnp.sum(acc_sc_ref[...], axis=0)
    counts = cnt_tc_ref[:, :1] + jnp.sum(cnt_sc_ref[...], axis=0)[:, :1]
    mean_x = sums / jnp.clip(counts, 1.0, None)
    h = lax.dot_general(mean_x, w1_ref[...],
                        (((1,), (1,)), ((), ())),
                        preferred_element_type=jnp.float32) + b1_ref[...]
    mu = jnp.mean(h, axis=0, keepdims=True)
    var = jnp.mean((h - mu) * (h - mu), axis=0, keepdims=True)
    h = (h - mu) / jnp.sqrt(var + EPS) * gamma_ref[...] + beta_ref[...]
    h = jnp.maximum(h, 0.0)
    o_ref[...] = jnp.dot(h, w2t_ref[...],
                         preferred_element_type=jnp.float32) + b2_ref[...]


def kernel(x, edge_index, edge_attr, u, batch, W1, b1, gamma, beta, W2, b2):
    batch3 = batch[:S_TC].reshape(S_TC // BLK, 1, BLK)

    acc_tc, cnt_tc = pl.pallas_call(
        _tc_sum_kernel,
        grid=(S_TC // BLK,),
        in_specs=[
            pl.BlockSpec((1, 1, BLK), lambda i: (i, 0, 0)),
            pl.BlockSpec((BLK, HIDDEN), lambda i: (i, 0)),
        ],
        out_specs=[
            pl.BlockSpec((NUM_GRAPHS, HIDDEN), lambda i: (0, 0)),
            pl.BlockSpec((NUM_GRAPHS, 128), lambda i: (0, 0)),
        ],
        out_shape=[
            jax.ShapeDtypeStruct((NUM_GRAPHS, HIDDEN), jnp.float32),
            jax.ShapeDtypeStruct((NUM_GRAPHS, 128), jnp.float32),
        ],
    )(batch3, x)

    sc_fn = pl.kernel(
        _sc_sum_kernel,
        out_type=[
            jax.ShapeDtypeStruct((NW, NUM_GRAPHS, HIDDEN), jnp.float32),
            jax.ShapeDtypeStruct((NW, NUM_GRAPHS, LANES), jnp.float32),
        ],
        mesh=plsc.VectorSubcoreMesh(core_axis_name="c", subcore_axis_name="s"),
        scratch_types=[
            pltpu.VMEM((2, CHUNK, HIDDEN), jnp.float32),
            pltpu.VMEM((NUM_GRAPHS, HIDDEN), jnp.float32),
            pltpu.VMEM((NUM_GRAPHS, LANES), jnp.float32),
            pltpu.VMEM((C_SC,), jnp.int32),
            pltpu.SemaphoreType.DMA,
            pltpu.SemaphoreType.DMA,
        ],
    )
    acc_sc, cnt_sc = sc_fn(x, batch)

    w2t = jnp.pad(W2.T, ((0, 0), (0, OPAD - OUTPUTS)))
    b2p = jnp.pad(b2, (0, OPAD - OUTPUTS)).reshape(1, OPAD)

    out = pl.pallas_call(
        _mlp_kernel,
        grid=(1,),
        in_specs=[
            pl.BlockSpec((NUM_GRAPHS, HIDDEN), lambda i: (0, 0)),
            pl.BlockSpec((NUM_GRAPHS, 128), lambda i: (0, 0)),
            pl.BlockSpec((NW, NUM_GRAPHS, HIDDEN), lambda i: (0, 0, 0)),
            pl.BlockSpec((NW, NUM_GRAPHS, LANES), lambda i: (0, 0, 0)),
            pl.BlockSpec((HIDDEN, HIDDEN), lambda i: (0, 0)),
            pl.BlockSpec((1, HIDDEN), lambda i: (0, 0)),
            pl.BlockSpec((1, HIDDEN), lambda i: (0, 0)),
            pl.BlockSpec((1, HIDDEN), lambda i: (0, 0)),
            pl.BlockSpec((HIDDEN, OPAD), lambda i: (0, 0)),
            pl.BlockSpec((1, OPAD), lambda i: (0, 0)),
        ],
        out_specs=pl.BlockSpec((NUM_GRAPHS, OPAD), lambda i: (0, 0)),
        out_shape=jax.ShapeDtypeStruct((NUM_GRAPHS, OPAD), jnp.float32),
    )(acc_tc, cnt_tc, acc_sc, cnt_sc, W1, b1.reshape(1, HIDDEN),
      gamma.reshape(1, HIDDEN), beta.reshape(1, HIDDEN), w2t, b2p)
    return out[:, :OUTPUTS]


# speedup vs baseline: 1.6912x; 1.6912x over previous
import functools

import jax
import jax.numpy as jnp
from jax import lax
from jax.experimental import pallas as pl
from jax.experimental.pallas import tpu as pltpu
from jax.experimental.pallas import tpu_sc as plsc

HIDDEN = 512
OUTPUTS = 2
NUM_GRAPHS = 64
EPS = 1e-5

NW = 32
C_SC = 160
SC_ROWS = NW * C_SC
S_TC = 10000 - SC_ROWS
BLK = S_TC // 2
CHUNK = 32
NCH = C_SC // CHUNK
LANES = 16
NCOL = HIDDEN // LANES
OPAD = 128


def _tc_sum_kernel(batch_ref, x_ref, acc_ref, cnt_ref):
    i = pl.program_id(0)

    @pl.when(i == 0)
    def _init():
        acc_ref[...] = jnp.zeros_like(acc_ref)
        cnt_ref[...] = jnp.zeros_like(cnt_ref)

    b = batch_ref[0, 0, :]
    gids = lax.broadcasted_iota(jnp.int32, (NUM_GRAPHS, BLK), 0)
    onehot = (b[None, :] == gids).astype(jnp.float32)
    acc_ref[...] += jnp.dot(onehot, x_ref[...],
                            preferred_element_type=jnp.float32)
    cnt_ref[...] = cnt_ref[...] + jnp.sum(onehot, axis=1, keepdims=True)


def _sc_sum_kernel(x_hbm, batch_hbm, acc_out, cnt_out,
                   rows_v, acc_v, cnt_v, idx_v, sem0, sem1):
    c = lax.axis_index("c")
    s = lax.axis_index("s")
    wid = c * 16 + s
    base = S_TC + wid * C_SC

    pltpu.sync_copy(batch_hbm.at[pl.ds(base, C_SC)], idx_v)

    zero = jnp.zeros((LANES,), jnp.float32)

    def zrow(g, carry):
        for k in range(NCOL):
            acc_v[g, pl.ds(k * LANES, LANES)] = zero
        cnt_v[g, :] = zero
        return carry

    lax.fori_loop(0, NUM_GRAPHS, zrow, 0)

    sems = [sem0, sem1]
    copies = [None, None]
    copies[0] = pltpu.make_async_copy(
        x_hbm.at[pl.ds(base, CHUNK)], rows_v.at[0], sems[0])
    copies[0].start()
    ones = jnp.ones((LANES,), jnp.float32)

    for ch in range(NCH):
        buf = ch % 2
        if ch + 1 < NCH:
            nbuf = (ch + 1) % 2
            copies[nbuf] = pltpu.make_async_copy(
                x_hbm.at[pl.ds(base + (ch + 1) * CHUNK, CHUNK)],
                rows_v.at[nbuf], sems[nbuf])
            copies[nbuf].start()
        copies[buf].wait()

        def group_body(g, carry, ch=ch, buf=buf):
            ids = idx_v[pl.ds(ch * CHUNK + g * LANES, LANES)]
            plsc.addupdate(cnt_v.at[0, :], ids.astype(jnp.float32))
            return carry

        lax.fori_loop(0, CHUNK // LANES, group_body, 0)

    pltpu.sync_copy(acc_v, acc_out.at[wid])
    pltpu.sync_copy(cnt_v, cnt_out.at[wid])


def _mlp_kernel(acc_tc_ref, cnt_tc_ref, acc_sc_ref, cnt_sc_ref,
                w1_ref, b1_ref, gamma_ref, beta_ref, w2t_ref, b2_ref, o_ref):
    sums = acc_tc_ref[...] + jnp.sum(acc_sc_ref[...], axis=0)
    counts = cnt_tc_ref[:, :1] + jnp.sum(cnt_sc_ref[...], axis=0)[:, :1]
    mean_x = sums / jnp.clip(counts, 1.0, None)
    h = lax.dot_general(mean_x, w1_ref[...],
                        (((1,), (1,)), ((), ())),
                        preferred_element_type=jnp.float32) + b1_ref[...]
    mu = jnp.mean(h, axis=0, keepdims=True)
    var = jnp.mean((h - mu) * (h - mu), axis=0, keepdims=True)
    h = (h - mu) / jnp.sqrt(var + EPS) * gamma_ref[...] + beta_ref[...]
    h = jnp.maximum(h, 0.0)
    o_ref[...] = jnp.dot(h, w2t_ref[...],
                         preferred_element_type=jnp.float32) + b2_ref[...]


def kernel(x, edge_index, edge_attr, u, batch, W1, b1, gamma, beta, W2, b2):
    batch3 = batch[:S_TC].reshape(S_TC // BLK, 1, BLK)

    acc_tc, cnt_tc = pl.pallas_call(
        _tc_sum_kernel,
        grid=(S_TC // BLK,),
        in_specs=[
            pl.BlockSpec((1, 1, BLK), lambda i: (i, 0, 0)),
            pl.BlockSpec((BLK, HIDDEN), lambda i: (i, 0)),
        ],
        out_specs=[
            pl.BlockSpec((NUM_GRAPHS, HIDDEN), lambda i: (0, 0)),
            pl.BlockSpec((NUM_GRAPHS, 128), lambda i: (0, 0)),
        ],
        out_shape=[
            jax.ShapeDtypeStruct((NUM_GRAPHS, HIDDEN), jnp.float32),
            jax.ShapeDtypeStruct((NUM_GRAPHS, 128), jnp.float32),
        ],
    )(batch3, x)

    sc_fn = pl.kernel(
        _sc_sum_kernel,
        out_type=[
            jax.ShapeDtypeStruct((NW, NUM_GRAPHS, HIDDEN), jnp.float32),
            jax.ShapeDtypeStruct((NW, NUM_GRAPHS, LANES), jnp.float32),
        ],
        mesh=plsc.VectorSubcoreMesh(core_axis_name="c", subcore_axis_name="s"),
        scratch_types=[
            pltpu.VMEM((2, CHUNK, HIDDEN), jnp.float32),
            pltpu.VMEM((NUM_GRAPHS, HIDDEN), jnp.float32),
            pltpu.VMEM((NUM_GRAPHS, LANES), jnp.float32),
            pltpu.VMEM((C_SC,), jnp.int32),
            pltpu.SemaphoreType.DMA,
            pltpu.SemaphoreType.DMA,
        ],
    )
    acc_sc, cnt_sc = sc_fn(x, batch)

    w2t = jnp.pad(W2.T, ((0, 0), (0, OPAD - OUTPUTS)))
    b2p = jnp.pad(b2, (0, OPAD - OUTPUTS)).reshape(1, OPAD)

    out = pl.pallas_call(
        _mlp_kernel,
        grid=(1,),
        in_specs=[
            pl.BlockSpec((NUM_GRAPHS, HIDDEN), lambda i: (0, 0)),
            pl.BlockSpec((NUM_GRAPHS, 128), lambda i: (0, 0)),
            pl.BlockSpec((NW, NUM_GRAPHS, HIDDEN), lambda i: (0, 0, 0)),
            pl.BlockSpec((NW, NUM_GRAPHS, LANES), lambda i: (0, 0, 0)),
            pl.BlockSpec((HIDDEN, HIDDEN), lambda i: (0, 0)),
            pl.BlockSpec((1, HIDDEN), lambda i: (0, 0)),
            pl.BlockSpec((1, HIDDEN), lambda i: (0, 0)),
            pl.BlockSpec((1, HIDDEN), lambda i: (0, 0)),
            pl.BlockSpec((HIDDEN, OPAD), lambda i: (0, 0)),
            pl.BlockSpec((1, OPAD), lambda i: (0, 0)),
        ],
        out_specs=pl.BlockSpec((NUM_GRAPHS, OPAD), lambda i: (0, 0)),
        out_shape=jax.ShapeDtypeStruct((NUM_GRAPHS, OPAD), jnp.float32),
    )(acc_tc, cnt_tc, acc_sc, cnt_sc, W1, b1.reshape(1, HIDDEN),
      gamma.reshape(1, HIDDEN), beta.reshape(1, HIDDEN), w2t, b2p)
    return out[:, :OUTPUTS]


# speedup vs baseline: 3.8458x; 2.2741x over previous
import functools

import jax
import jax.numpy as jnp
from jax.experimental import pallas as pl
from jax.experimental.pallas import tpu as pltpu

HIDDEN = 512
HALF = HIDDEN // 2
OUTPUTS = 2
NUM_GRAPHS = 64
EPS = 1e-5

BLK = 5000
OPAD = 128


def _fused_kernel(batch_ref, x_ref, w1_ref, b1_ref, gamma_ref,
                  beta_ref, w2t_ref, b2_ref, o_ref, acc_ref, cnt_ref, *,
                  nblocks):
    i = pl.program_id(0)

    @pl.when(i == 0)
    def _init():
        acc_ref[...] = jnp.zeros_like(acc_ref)
        cnt_ref[...] = jnp.zeros_like(cnt_ref)

    b = batch_ref[0, 0, :]
    gids = jax.lax.broadcasted_iota(jnp.int32, (NUM_GRAPHS, BLK), 0)
    onehot = (b[None, :] == gids).astype(jnp.float32)
    acc_ref[...] += jnp.dot(onehot, x_ref[...],
                            preferred_element_type=jnp.float32)
    cnt_ref[...] = cnt_ref[...] + jnp.sum(onehot, axis=1, keepdims=True)

    @pl.when(i == nblocks - 1)
    def _finish():
        counts = jnp.clip(cnt_ref[:, :1], 1.0, None)
        mean_x = acc_ref[...] / counts
        h = jax.lax.dot_general(mean_x, w1_ref[...],
                                (((1,), (1,)), ((), ())),
                                preferred_element_type=jnp.float32) + b1_ref[...]
        mu = jnp.mean(h, axis=0, keepdims=True)
        var = jnp.mean((h - mu) * (h - mu), axis=0, keepdims=True)
        h = (h - mu) / jnp.sqrt(var + EPS) * gamma_ref[...] + beta_ref[...]
        h = jnp.maximum(h, 0.0)
        o_ref[...] = jnp.dot(h, w2t_ref[...],
                             preferred_element_type=jnp.float32) + b2_ref[...]


def kernel(x, edge_index, edge_attr, u, batch, W1, b1, gamma, beta, W2, b2):
    n = x.shape[0]
    nblocks = n // BLK
    batch3 = batch.reshape(nblocks, 1, BLK)

    w2t = jnp.pad(W2.T, ((0, 0), (0, OPAD - OUTPUTS)))
    b2p = jnp.pad(b2, (0, OPAD - OUTPUTS)).reshape(1, OPAD)

    out = pl.pallas_call(
        functools.partial(_fused_kernel, nblocks=nblocks),
        grid=(nblocks,),
        in_specs=[
            pl.BlockSpec((1, 1, BLK), lambda i: (i, 0, 0)),
            pl.BlockSpec((BLK, HIDDEN), lambda i: (i, 0)),
            pl.BlockSpec((HIDDEN, HIDDEN), lambda i: (0, 0)),
            pl.BlockSpec((1, HIDDEN), lambda i: (0, 0)),
            pl.BlockSpec((1, HIDDEN), lambda i: (0, 0)),
            pl.BlockSpec((1, HIDDEN), lambda i: (0, 0)),
            pl.BlockSpec((HIDDEN, OPAD), lambda i: (0, 0)),
            pl.BlockSpec((1, OPAD), lambda i: (0, 0)),
        ],
        out_specs=pl.BlockSpec((NUM_GRAPHS, OPAD), lambda i: (0, 0)),
        out_shape=jax.ShapeDtypeStruct((NUM_GRAPHS, OPAD), jnp.float32),
        scratch_shapes=[
            pltpu.VMEM((NUM_GRAPHS, HIDDEN), jnp.float32),
            pltpu.VMEM((NUM_GRAPHS, 128), jnp.float32),
        ],
    )(batch3, x, W1, b1.reshape(1, HIDDEN), gamma.reshape(1, HIDDEN),
      beta.reshape(1, HIDDEN), w2t, b2p)
    return out[:, :OUTPUTS]


# speedup vs baseline: 3.8948x; 1.0127x over previous
import functools

import jax
import jax.numpy as jnp
from jax.experimental import pallas as pl
from jax.experimental.pallas import tpu as pltpu

HIDDEN = 512
HALF = HIDDEN // 2
OUTPUTS = 2
NUM_GRAPHS = 64
EPS = 1e-5

BLK = 5000
OPAD = 128


def _fused_kernel(batch_ref, xlo_ref, xhi_ref, w1_ref, b1_ref, gamma_ref,
                  beta_ref, w2t_ref, b2_ref, o_ref, acc_ref, cnt_ref, *,
                  nblocks):
    i = pl.program_id(0)

    @pl.when(i == 0)
    def _init():
        acc_ref[...] = jnp.zeros_like(acc_ref)
        cnt_ref[...] = jnp.zeros_like(cnt_ref)

    b = batch_ref[0, 0, :]
    gids = jax.lax.broadcasted_iota(jnp.int32, (NUM_GRAPHS, BLK), 0)
    onehot = (b[None, :] == gids).astype(jnp.float32)
    acc_ref[:, :HALF] += jnp.dot(onehot, xlo_ref[...],
                                 preferred_element_type=jnp.float32)
    acc_ref[:, HALF:] += jnp.dot(onehot, xhi_ref[...],
                                 preferred_element_type=jnp.float32)
    cnt_ref[...] = cnt_ref[...] + jnp.sum(onehot, axis=1, keepdims=True)

    @pl.when(i == nblocks - 1)
    def _finish():
        counts = jnp.clip(cnt_ref[:, :1], 1.0, None)
        mean_x = acc_ref[...] / counts
        h = jax.lax.dot_general(mean_x, w1_ref[...],
                                (((1,), (1,)), ((), ())),
                                preferred_element_type=jnp.float32) + b1_ref[...]
        mu = jnp.mean(h, axis=0, keepdims=True)
        var = jnp.mean((h - mu) * (h - mu), axis=0, keepdims=True)
        h = (h - mu) / jnp.sqrt(var + EPS) * gamma_ref[...] + beta_ref[...]
        h = jnp.maximum(h, 0.0)
        o_ref[...] = jnp.dot(h, w2t_ref[...],
                             preferred_element_type=jnp.float32) + b2_ref[...]


def kernel(x, edge_index, edge_attr, u, batch, W1, b1, gamma, beta, W2, b2):
    n = x.shape[0]
    nblocks = n // BLK
    batch3 = batch.reshape(nblocks, 1, BLK)

    w2t = jnp.pad(W2.T, ((0, 0), (0, OPAD - OUTPUTS)))
    b2p = jnp.pad(b2, (0, OPAD - OUTPUTS)).reshape(1, OPAD)

    out = pl.pallas_call(
        functools.partial(_fused_kernel, nblocks=nblocks),
        grid=(nblocks,),
        in_specs=[
            pl.BlockSpec((1, 1, BLK), lambda i: (i, 0, 0)),
            pl.BlockSpec((BLK, HALF), lambda i: (i, 0)),
            pl.BlockSpec((BLK, HALF), lambda i: (i, 1)),
            pl.BlockSpec((HIDDEN, HIDDEN), lambda i: (0, 0)),
            pl.BlockSpec((1, HIDDEN), lambda i: (0, 0)),
            pl.BlockSpec((1, HIDDEN), lambda i: (0, 0)),
            pl.BlockSpec((1, HIDDEN), lambda i: (0, 0)),
            pl.BlockSpec((HIDDEN, OPAD), lambda i: (0, 0)),
            pl.BlockSpec((1, OPAD), lambda i: (0, 0)),
        ],
        out_specs=pl.BlockSpec((NUM_GRAPHS, OPAD), lambda i: (0, 0)),
        out_shape=jax.ShapeDtypeStruct((NUM_GRAPHS, OPAD), jnp.float32),
        scratch_shapes=[
            pltpu.VMEM((NUM_GRAPHS, HIDDEN), jnp.float32),
            pltpu.VMEM((NUM_GRAPHS, 128), jnp.float32),
        ],
    )(batch3, x, x, W1, b1.reshape(1, HIDDEN), gamma.reshape(1, HIDDEN),
      beta.reshape(1, HIDDEN), w2t, b2p)
    return out[:, :OUTPUTS]
